# baseline (speedup 1.0000x reference)
import functools

import jax
import jax.numpy as jnp
from jax import lax
from jax.experimental import pallas as pl
from jax.experimental.pallas import tpu as pltpu
from jax.experimental.pallas import tpu_sc as plsc

_NC = 2
_NS = 16
_NW = _NC * _NS
_CHUNK = 128
_NBUF = 4


@functools.lru_cache(maxsize=None)
def _build_gather(n_s: int, d: int):
    mesh = plsc.VectorSubcoreMesh(core_axis_name="c", subcore_axis_name="s")
    n_dt = d // 8
    tile_words = 8 * 128

    @functools.partial(
        pl.kernel,
        mesh=mesh,
        out_type=jax.ShapeDtypeStruct((n_s, n_dt, _NW, tile_words), jnp.float32),
        scratch_types=[
            pltpu.VMEM((n_s, _CHUNK), jnp.int32),
            pltpu.VMEM((_NBUF, _CHUNK, d), jnp.float32),
            pltpu.VMEM((_CHUNK * d,), jnp.float32),
            pltpu.VMEM((_CHUNK * d,), jnp.float32),
            pltpu.SemaphoreType.DMA((_NBUF,)),
            pltpu.SemaphoreType.DMA((2,)),
        ],
        compiler_params=pltpu.CompilerParams(
            use_tc_tiling_on_sc=False, needs_layout_passes=False),
    )
    def gather_kernel(idx_hbm, table_hbm, out_hbm, idx_v, rows_v, trans_a,
                      trans_b, gsems, osems):
        trans_refs = (trans_a, trans_b)
        wid = lax.axis_index("s") * _NC + lax.axis_index("c")
        pltpu.sync_copy(idx_hbm.at[:, wid], idx_v)
        v128 = lax.iota(jnp.int32, 16) * 128

        def gather(t, b):
            return pltpu.make_async_copy(
                table_hbm.at[idx_v.at[t]], rows_v.at[b], gsems.at[b])

        def out_copy(t, ts, dt):
            return pltpu.make_async_copy(
                trans_refs[ts].at[pl.ds(tile_words * dt, tile_words)],
                out_hbm.at[t, dt, wid], osems.at[ts])

        def transpose_chunk(b, ts):
            @plsc.parallel_loop(0, _CHUNK, step=1, unroll=8)
            def body(t):
                for h in range(d // 16):
                    vals = rows_v[b, t, pl.ds(16 * h, 16)]
                    plsc.store_scatter(
                        trans_refs[ts], [v128 + (t + 16 * 128 * h)], vals)

        for b in range(_NBUF):
            gather(b, b).start()

        def outer(gi, carry):
            g = gi * _NBUF
            for b in range(_NBUF):
                t = g + b
                ts = b % 2
                gather(t, b).wait()
                @pl.when(t >= 2)
                def _():
                    for dt in range(n_dt):
                        out_copy(t - 2, ts, dt).wait()
                transpose_chunk(b, ts)
                for dt in range(n_dt):
                    out_copy(t, ts, dt).start()
                gather(t + _NBUF, b).start()
            return carry

        lax.fori_loop(0, n_s // _NBUF - 1, outer, 0)

        for b in range(_NBUF):
            t = n_s - _NBUF + b
            ts = b % 2
            gather(t, b).wait()
            for dt in range(n_dt):
                out_copy(t - 2, ts, dt).wait()
            transpose_chunk(b, ts)
            for dt in range(n_dt):
                out_copy(t, ts, dt).start()
        for b in range(_NBUF - 2, _NBUF):
            t = n_s - _NBUF + b
            for dt in range(n_dt):
                out_copy(t, b % 2, dt).wait()

    return gather_kernel


_RNB = 4


@functools.lru_cache(maxsize=None)
def _build_table_reformat(v: int, d: int):
    mesh = plsc.VectorSubcoreMesh(core_axis_name="c", subcore_axis_name="s")
    n_full = v // 128
    tail = v - 128 * n_full
    last_col = 128 * (n_full - 1)
    trips = -(-n_full // (_NW * _RNB)) * _RNB
    cw = 128 * d

    @functools.partial(
        pl.kernel,
        mesh=mesh,
        out_type=jax.ShapeDtypeStruct((v * d,), jnp.float32),
        scratch_types=[
            pltpu.VMEM((_RNB, d, 128), jnp.float32),
            pltpu.VMEM((cw,), jnp.float32),
            pltpu.VMEM((cw,), jnp.float32),
            pltpu.VMEM((d, 64), jnp.float32),
            pltpu.SemaphoreType.DMA((_RNB,)),
            pltpu.SemaphoreType.DMA((2,)),
        ],
        compiler_params=pltpu.CompilerParams(
            use_tc_tiling_on_sc=True, needs_layout_passes=False),
    )
    def reformat_kernel(wt_hbm, out_hbm, in_v, trans_a, trans_b, tail_v,
                        isems, osems):
        trans_refs = (trans_a, trans_b)
        wid = lax.axis_index("s") * _NC + lax.axis_index("c")
        v32 = lax.iota(jnp.int32, 16) * d

        def col0_of(k):
            return pl.multiple_of(
                jnp.minimum(128 * (wid + k * _NW), last_col), 128)

        def in_copy(k, b):
            return pltpu.make_async_copy(
                wt_hbm.at[:, pl.ds(col0_of(k), 128)], in_v.at[b], isems.at[b])

        def out_copy(k, ts):
            return pltpu.make_async_copy(
                trans_refs[ts], out_hbm.at[pl.ds(col0_of(k) * d, cw)],
                osems.at[ts])

        def transpose_chunk(b, ts):
            @plsc.parallel_loop(0, d * 8, step=1, unroll=8)
            def body(i):
                dd = i >> 3
                g = i & 7
                vals = in_v[b, dd, pl.ds(16 * (i & 7), 16)]
                plsc.store_scatter(trans_refs[ts], [v32 + (16 * d * g + dd)],
                                   vals)

        if tail:
            @pl.when(wid == 0)
            def _():
                pltpu.sync_copy(wt_hbm.at[:, pl.ds(128 * n_full, tail)],
                                tail_v)
                for dd_g in range(d * (tail // 16)):
                    dd, g = dd_g // (tail // 16), dd_g % (tail // 16)
                    vals = tail_v[dd, pl.ds(16 * g, 16)]
                    plsc.store_scatter(trans_a, [v32 + (16 * d * g + dd)],
                                       vals)
                pltpu.sync_copy(trans_a.at[pl.ds(0, tail * d)],
                                out_hbm.at[pl.ds(128 * n_full * d, tail * d)])

        for b in range(_RNB):
            in_copy(b, b).start()

        def outer(gi, carry):
            for b in range(_RNB):
                k = gi * _RNB + b
                ts = b % 2
                in_copy(k, b).wait()
                @pl.when(k >= 2)
                def _():
                    out_copy(k - 2, ts).wait()
                transpose_chunk(b, ts)
                out_copy(k, ts).start()
                in_copy(k + _RNB, b).start()
            return carry

        lax.fori_loop(0, trips // _RNB - 1, outer, 0)

        for b in range(_RNB):
            k = trips - _RNB + b
            ts = b % 2
            in_copy(k, b).wait()
            out_copy(k - 2, ts).wait()
            transpose_chunk(b, ts)
            out_copy(k, ts).start()
        for b in range(_RNB - 2, _RNB):
            out_copy(trips - _RNB + b, b % 2).wait()

    return reformat_kernel


def kernel(token_ids, weight):
    bsz, seq = token_ids.shape
    d = weight.shape[1]
    v = weight.shape[0]
    assert bsz == _NW * _CHUNK and d % 16 == 0
    idx3 = token_ids.T.reshape(seq, _NW, _CHUNK).astype(jnp.int32)
    table = _build_table_reformat(v, d)(weight.T).reshape(v, d)
    out6 = _build_gather(seq, d)(idx3, table)
    out5 = out6.reshape(seq, d // 8, _NW, 8, 128)
    return out5.transpose(2, 4, 0, 1, 3).reshape(bsz, seq, d)


# speedup vs baseline: 1.0529x; 1.0529x over previous
import functools

import jax
import jax.numpy as jnp
from jax import lax
from jax.experimental import pallas as pl
from jax.experimental.pallas import tpu as pltpu
from jax.experimental.pallas import tpu_sc as plsc

_NC = 2
_NS = 16
_NW = _NC * _NS
_CHUNK = 128
_NBUF = 4


@functools.lru_cache(maxsize=None)
def _build_gather(n_s: int, d: int):
    mesh = plsc.VectorSubcoreMesh(core_axis_name="c", subcore_axis_name="s")
    n_dt = d // 8
    tile_words = 8 * 128

    @functools.partial(
        pl.kernel,
        mesh=mesh,
        out_type=jax.ShapeDtypeStruct((n_s, n_dt, _NW, tile_words), jnp.float32),
        scratch_types=[
            pltpu.VMEM((n_s, _CHUNK), jnp.int32),
            pltpu.VMEM((_NBUF, _CHUNK, d), jnp.float32),
            pltpu.VMEM((_CHUNK * d,), jnp.float32),
            pltpu.VMEM((_CHUNK * d,), jnp.float32),
            pltpu.SemaphoreType.DMA((_NBUF,)),
            pltpu.SemaphoreType.DMA((2,)),
        ],
        compiler_params=pltpu.CompilerParams(
            use_tc_tiling_on_sc=False, needs_layout_passes=False),
    )
    def gather_kernel(idx_hbm, table_hbm, out_hbm, idx_v, rows_v, trans_a,
                      trans_b, gsems, osems):
        trans_refs = (trans_a, trans_b)
        wid = lax.axis_index("s") * _NC + lax.axis_index("c")
        pltpu.sync_copy(idx_hbm.at[:, wid], idx_v)
        v128 = lax.iota(jnp.int32, 16) * 128

        def gather(t, b):
            return pltpu.make_async_copy(
                table_hbm.at[idx_v.at[t]], rows_v.at[b], gsems.at[b])

        def out_copy(t, ts, dt):
            return pltpu.make_async_copy(
                trans_refs[ts].at[pl.ds(tile_words * dt, tile_words)],
                out_hbm.at[t, dt, wid], osems.at[ts])

        def transpose_chunk(b, ts):
            @plsc.parallel_loop(0, _CHUNK, step=1, unroll=16)
            def body(t):
                for h in range(d // 16):
                    vals = rows_v[b, t, pl.ds(16 * h, 16)]
                    plsc.store_scatter(
                        trans_refs[ts], [v128 + (t + 16 * 128 * h)], vals)

        for b in range(_NBUF):
            gather(b, b).start()

        def outer(gi, carry):
            g = gi * _NBUF
            for b in range(_NBUF):
                t = g + b
                ts = b % 2
                gather(t, b).wait()
                @pl.when(t >= 2)
                def _():
                    for dt in range(n_dt):
                        out_copy(t - 2, ts, dt).wait()
                transpose_chunk(b, ts)
                for dt in range(n_dt):
                    out_copy(t, ts, dt).start()
                gather(t + _NBUF, b).start()
            return carry

        lax.fori_loop(0, n_s // _NBUF - 1, outer, 0)

        for b in range(_NBUF):
            t = n_s - _NBUF + b
            ts = b % 2
            gather(t, b).wait()
            for dt in range(n_dt):
                out_copy(t - 2, ts, dt).wait()
            transpose_chunk(b, ts)
            for dt in range(n_dt):
                out_copy(t, ts, dt).start()
        for b in range(_NBUF - 2, _NBUF):
            t = n_s - _NBUF + b
            for dt in range(n_dt):
                out_copy(t, b % 2, dt).wait()

    return gather_kernel


_RNB = 4


@functools.lru_cache(maxsize=None)
def _build_table_reformat(v: int, d: int):
    mesh = plsc.VectorSubcoreMesh(core_axis_name="c", subcore_axis_name="s")
    n_full = v // 128
    tail = v - 128 * n_full
    last_col = 128 * (n_full - 1)
    trips = -(-n_full // (_NW * _RNB)) * _RNB
    cw = 128 * d

    @functools.partial(
        pl.kernel,
        mesh=mesh,
        out_type=jax.ShapeDtypeStruct((v * d,), jnp.float32),
        scratch_types=[
            pltpu.VMEM((_RNB, d, 128), jnp.float32),
            pltpu.VMEM((cw,), jnp.float32),
            pltpu.VMEM((cw,), jnp.float32),
            pltpu.VMEM((d, 64), jnp.float32),
            pltpu.SemaphoreType.DMA((_RNB,)),
            pltpu.SemaphoreType.DMA((2,)),
        ],
        compiler_params=pltpu.CompilerParams(
            use_tc_tiling_on_sc=True, needs_layout_passes=False),
    )
    def reformat_kernel(wt_hbm, out_hbm, in_v, trans_a, trans_b, tail_v,
                        isems, osems):
        trans_refs = (trans_a, trans_b)
        wid = lax.axis_index("s") * _NC + lax.axis_index("c")
        v32 = lax.iota(jnp.int32, 16) * d

        def col0_of(k):
            return pl.multiple_of(
                jnp.minimum(128 * (wid + k * _NW), last_col), 128)

        def in_copy(k, b):
            return pltpu.make_async_copy(
                wt_hbm.at[:, pl.ds(col0_of(k), 128)], in_v.at[b], isems.at[b])

        def out_copy(k, ts):
            return pltpu.make_async_copy(
                trans_refs[ts], out_hbm.at[pl.ds(col0_of(k) * d, cw)],
                osems.at[ts])

        def transpose_chunk(b, ts):
            @plsc.parallel_loop(0, 8, step=1, unroll=2)
            def body(g):
                base = v32 + 16 * d * g
                for dd in range(d):
                    vals = in_v[b, dd, pl.ds(16 * g, 16)]
                    plsc.store_scatter(trans_refs[ts], [base + dd], vals)

        if tail:
            @pl.when(wid == 0)
            def _():
                pltpu.sync_copy(wt_hbm.at[:, pl.ds(128 * n_full, tail)],
                                tail_v)
                for dd_g in range(d * (tail // 16)):
                    dd, g = dd_g // (tail // 16), dd_g % (tail // 16)
                    vals = tail_v[dd, pl.ds(16 * g, 16)]
                    plsc.store_scatter(trans_a, [v32 + (16 * d * g + dd)],
                                       vals)
                pltpu.sync_copy(trans_a.at[pl.ds(0, tail * d)],
                                out_hbm.at[pl.ds(128 * n_full * d, tail * d)])

        for b in range(_RNB):
            in_copy(b, b).start()

        def outer(gi, carry):
            for b in range(_RNB):
                k = gi * _RNB + b
                ts = b % 2
                in_copy(k, b).wait()
                @pl.when(k >= 2)
                def _():
                    out_copy(k - 2, ts).wait()
                transpose_chunk(b, ts)
                out_copy(k, ts).start()
                in_copy(k + _RNB, b).start()
            return carry

        lax.fori_loop(0, trips // _RNB - 1, outer, 0)

        for b in range(_RNB):
            k = trips - _RNB + b
            ts = b % 2
            in_copy(k, b).wait()
            out_copy(k - 2, ts).wait()
            transpose_chunk(b, ts)
            out_copy(k, ts).start()
        for b in range(_RNB - 2, _RNB):
            out_copy(trips - _RNB + b, b % 2).wait()

    return reformat_kernel


def kernel(token_ids, weight):
    bsz, seq = token_ids.shape
    d = weight.shape[1]
    v = weight.shape[0]
    assert bsz == _NW * _CHUNK and d % 16 == 0
    idx3 = token_ids.T.reshape(seq, _NW, _CHUNK).astype(jnp.int32)
    table = _build_table_reformat(v, d)(weight.T).reshape(v, d)
    out6 = _build_gather(seq, d)(idx3, table)
    out5 = out6.reshape(seq, d // 8, _NW, 8, 128)
    return out5.transpose(2, 4, 0, 1, 3).reshape(bsz, seq, d)


# speedup vs baseline: 1.6476x; 1.5649x over previous
import functools

import jax
import jax.numpy as jnp
from jax import lax
from jax.experimental import pallas as pl
from jax.experimental.pallas import tpu as pltpu
from jax.experimental.pallas import tpu_sc as plsc

_NC = 2
_NS = 16
_NW = _NC * _NS
_CHUNK = 128
_NBUF = 4


@functools.lru_cache(maxsize=None)
def _build_gather(n_s: int, d: int):
    mesh = plsc.VectorSubcoreMesh(core_axis_name="c", subcore_axis_name="s")
    n_dt = d // 8
    tile_words = 8 * 128

    @functools.partial(
        pl.kernel,
        mesh=mesh,
        out_type=jax.ShapeDtypeStruct((n_s, n_dt, _NW, 8, 128), jnp.float32),
        scratch_types=[
            pltpu.VMEM((n_s, _CHUNK), jnp.int32),
            pltpu.VMEM((_NBUF, _CHUNK, d), jnp.float32),
            pltpu.VMEM((d, 129), jnp.float32),
            pltpu.VMEM((d, 129), jnp.float32),
            pltpu.SemaphoreType.DMA((_NBUF,)),
            pltpu.SemaphoreType.DMA((2,)),
        ],
        compiler_params=pltpu.CompilerParams(
            use_tc_tiling_on_sc=False, needs_layout_passes=False),
    )
    def gather_kernel(idx_hbm, table_hbm, out_hbm, idx_v, rows_v, trans_a,
                      trans_b, gsems, osems):
        trans_refs = (trans_a, trans_b)
        wid = lax.axis_index("s") * _NC + lax.axis_index("c")
        pltpu.sync_copy(idx_hbm.at[:, wid], idx_v)
        v16 = lax.iota(jnp.int32, 16)

        def gather(t, b):
            return pltpu.make_async_copy(
                table_hbm.at[idx_v.at[t]], rows_v.at[b], gsems.at[b])

        def out_copy(t, ts, dt):
            return pltpu.make_async_copy(
                trans_refs[ts].at[pl.ds(8 * dt, 8), pl.ds(0, 128)],
                out_hbm.at[t, dt, wid], osems.at[ts])

        def transpose_chunk(b, ts):
            @plsc.parallel_loop(0, _CHUNK, step=1, unroll=16)
            def body(t):
                col = jnp.full((16,), t, jnp.int32)
                for h in range(d // 16):
                    vals = rows_v[b, t, pl.ds(16 * h, 16)]
                    plsc.store_scatter(
                        trans_refs[ts], [v16 + 16 * h, col], vals)

        for b in range(_NBUF):
            gather(b, b).start()

        def outer(gi, carry):
            g = gi * _NBUF
            for b in range(_NBUF):
                t = g + b
                ts = b % 2
                gather(t, b).wait()
                @pl.when(t >= 2)
                def _():
                    for dt in range(n_dt):
                        out_copy(t - 2, ts, dt).wait()
                transpose_chunk(b, ts)
                for dt in range(n_dt):
                    out_copy(t, ts, dt).start()
                gather(t + _NBUF, b).start()
            return carry

        lax.fori_loop(0, n_s // _NBUF - 1, outer, 0)

        for b in range(_NBUF):
            t = n_s - _NBUF + b
            ts = b % 2
            gather(t, b).wait()
            for dt in range(n_dt):
                out_copy(t - 2, ts, dt).wait()
            transpose_chunk(b, ts)
            for dt in range(n_dt):
                out_copy(t, ts, dt).start()
        for b in range(_NBUF - 2, _NBUF):
            t = n_s - _NBUF + b
            for dt in range(n_dt):
                out_copy(t, b % 2, dt).wait()

    return gather_kernel


_RNB = 4


@functools.lru_cache(maxsize=None)
def _build_table_reformat(v: int, d: int):
    mesh = plsc.VectorSubcoreMesh(core_axis_name="c", subcore_axis_name="s")
    n_full = v // 128
    tail = v - 128 * n_full
    last_col = 128 * (n_full - 1)
    trips = -(-n_full // (_NW * _RNB)) * _RNB
    cw = 128 * d

    @functools.partial(
        pl.kernel,
        mesh=mesh,
        out_type=jax.ShapeDtypeStruct((v * d,), jnp.float32),
        scratch_types=[
            pltpu.VMEM((_RNB, d, 128), jnp.float32),
            pltpu.VMEM((cw,), jnp.float32),
            pltpu.VMEM((cw,), jnp.float32),
            pltpu.VMEM((d, 64), jnp.float32),
            pltpu.SemaphoreType.DMA((_RNB,)),
            pltpu.SemaphoreType.DMA((2,)),
        ],
        compiler_params=pltpu.CompilerParams(
            use_tc_tiling_on_sc=True, needs_layout_passes=False),
    )
    def reformat_kernel(wt_hbm, out_hbm, in_v, trans_a, trans_b, tail_v,
                        isems, osems):
        trans_refs = (trans_a, trans_b)
        wid = lax.axis_index("s") * _NC + lax.axis_index("c")
        v32 = lax.iota(jnp.int32, 16) * d

        def col0_of(k):
            return pl.multiple_of(
                jnp.minimum(128 * (wid + k * _NW), last_col), 128)

        def in_copy(k, b):
            return pltpu.make_async_copy(
                wt_hbm.at[:, pl.ds(col0_of(k), 128)], in_v.at[b], isems.at[b])

        def out_copy(k, ts):
            return pltpu.make_async_copy(
                trans_refs[ts], out_hbm.at[pl.ds(col0_of(k) * d, cw)],
                osems.at[ts])

        def transpose_chunk(b, ts):
            @plsc.parallel_loop(0, 8, step=1, unroll=2)
            def body(g):
                base = v32 + 16 * d * g
                for dd in range(d):
                    vals = in_v[b, dd, pl.ds(16 * g, 16)]
                    plsc.store_scatter(trans_refs[ts], [base + dd], vals)

        if tail:
            @pl.when(wid == 0)
            def _():
                pltpu.sync_copy(wt_hbm.at[:, pl.ds(128 * n_full, tail)],
                                tail_v)
                for dd_g in range(d * (tail // 16)):
                    dd, g = dd_g // (tail // 16), dd_g % (tail // 16)
                    vals = tail_v[dd, pl.ds(16 * g, 16)]
                    plsc.store_scatter(trans_a, [v32 + (16 * d * g + dd)],
                                       vals)
                pltpu.sync_copy(trans_a.at[pl.ds(0, tail * d)],
                                out_hbm.at[pl.ds(128 * n_full * d, tail * d)])

        for b in range(_RNB):
            in_copy(b, b).start()

        def outer(gi, carry):
            for b in range(_RNB):
                k = gi * _RNB + b
                ts = b % 2
                in_copy(k, b).wait()
                @pl.when(k >= 2)
                def _():
                    out_copy(k - 2, ts).wait()
                transpose_chunk(b, ts)
                out_copy(k, ts).start()
                in_copy(k + _RNB, b).start()
            return carry

        lax.fori_loop(0, trips // _RNB - 1, outer, 0)

        for b in range(_RNB):
            k = trips - _RNB + b
            ts = b % 2
            in_copy(k, b).wait()
            out_copy(k - 2, ts).wait()
            transpose_chunk(b, ts)
            out_copy(k, ts).start()
        for b in range(_RNB - 2, _RNB):
            out_copy(trips - _RNB + b, b % 2).wait()

    return reformat_kernel


def kernel(token_ids, weight):
    bsz, seq = token_ids.shape
    d = weight.shape[1]
    v = weight.shape[0]
    assert bsz == _NW * _CHUNK and d % 16 == 0
    idx3 = token_ids.T.reshape(seq, _NW, _CHUNK).astype(jnp.int32)
    out6 = _build_gather(seq, d)(idx3, weight)
    out5 = out6.reshape(seq, d // 8, _NW, 8, 128)
    return out5.transpose(2, 4, 0, 1, 3).reshape(bsz, seq, d)
